# baseline (speedup 1.0000x reference)
import functools
import numpy as np
import jax
import jax.numpy as jnp
from jax import lax
from jax.experimental import pallas as pl
from jax.experimental.pallas import tpu as pltpu

_NODE_TYPES = ('paper', 'author')
_EDGE_TYPES = (('paper', 'cites', 'paper'), ('author', 'writes', 'paper'), ('paper', 'rev_writes', 'author'))
_H = 4
_HID = 128
_DH = _HID // _H
_L = 3


def _jk_matmul_kernel(h0_ref, h1_ref, h2_ref, w0_ref, w1_ref, w2_ref, b_ref, o_ref):
    acc = jnp.dot(h0_ref[...], w0_ref[...], preferred_element_type=jnp.float32)
    acc += jnp.dot(h1_ref[...], w1_ref[...], preferred_element_type=jnp.float32)
    acc += jnp.dot(h2_ref[...], w2_ref[...], preferred_element_type=jnp.float32)
    o_ref[...] = acc + b_ref[...]


def _jk_project(hs, Wout, bout):
    n = hs[0].shape[0]
    out_dim = Wout.shape[1]
    w0, w1, w2 = Wout[:_HID], Wout[_HID:2 * _HID], Wout[2 * _HID:]
    blk = 1000
    grid = (n // blk,)
    return pl.pallas_call(
        _jk_matmul_kernel,
        grid=grid,
        in_specs=[
            pl.BlockSpec((blk, _HID), lambda i: (i, 0)),
            pl.BlockSpec((blk, _HID), lambda i: (i, 0)),
            pl.BlockSpec((blk, _HID), lambda i: (i, 0)),
            pl.BlockSpec((_HID, out_dim), lambda i: (0, 0)),
            pl.BlockSpec((_HID, out_dim), lambda i: (0, 0)),
            pl.BlockSpec((_HID, out_dim), lambda i: (0, 0)),
            pl.BlockSpec((1, out_dim), lambda i: (0, 0)),
        ],
        out_specs=pl.BlockSpec((blk, out_dim), lambda i: (i, 0)),
        out_shape=jax.ShapeDtypeStruct((n, out_dim), jnp.float32),
    )(hs[0], hs[1], hs[2], w0, w1, w2, bout[None, :])


def _segment_softmax(scores, seg, num_seg):
    m = jax.ops.segment_max(scores, seg, num_segments=num_seg)
    m = jnp.where(jnp.isneginf(m), 0.0, m)
    e = jnp.exp(scores - m[seg])
    den = jax.ops.segment_sum(e, seg, num_segments=num_seg)
    return e / (den[seg] + 1e-16)


def kernel(x_paper, x_author, edge_index_cites, edge_index_writes, edge_index_rev_writes, batch_paper, batch_author, params):
    xd = {'paper': x_paper, 'author': x_author}
    eid = {'cites': edge_index_cites, 'writes': edge_index_writes, 'rev_writes': edge_index_rev_writes}
    x = dict(xd)
    xs = {nt: [] for nt in _NODE_TYPES}
    for l in range(_L):
        lp = params['layers'][l]
        k = {nt: (x[nt] @ lp['Wk'][nt]).reshape(-1, _H, _DH) for nt in _NODE_TYPES}
        q = {nt: (x[nt] @ lp['Wq'][nt]).reshape(-1, _H, _DH) for nt in _NODE_TYPES}
        v = {nt: (x[nt] @ lp['Wv'][nt]).reshape(-1, _H, _DH) for nt in _NODE_TYPES}
        agg = {nt: jnp.zeros((x[nt].shape[0], _H, _DH), jnp.float32) for nt in _NODE_TYPES}
        for (src_t, rel, dst_t) in _EDGE_TYPES:
            ei = eid[rel]
            src, dst = ei[0], ei[1]
            k_rel = jnp.einsum('nhd,hde->nhe', k[src_t], lp['a_rel'][rel])
            v_rel = jnp.einsum('nhd,hde->nhe', v[src_t], lp['m_rel'][rel])
            k_e = k_rel[src]
            v_e = v_rel[src]
            q_e = q[dst_t][dst]
            n_dst = x[dst_t].shape[0]
            alpha = (q_e * k_e).sum(-1) * lp['p_rel'][rel][None, :] / np.sqrt(_DH)
            alpha = _segment_softmax(alpha, dst, n_dst)
            agg[dst_t] = agg[dst_t] + jax.ops.segment_sum(v_e * alpha[:, :, None], dst, num_segments=n_dst)
        new_x = {}
        for nt in _NODE_TYPES:
            o = jax.nn.gelu(agg[nt].reshape(-1, _HID)) @ lp['Wa'][nt]
            beta = jax.nn.sigmoid(lp['skip'][nt])
            h = beta * o + (1.0 - beta) * x[nt]
            cnt = h.shape[0] * _HID
            mean = h.sum() / cnt
            hc = h - mean
            var = (hc * hc).sum() / cnt
            h = hc / jnp.sqrt(var + 1e-5) * lp['gamma'][nt] + lp['beta'][nt]
            new_x[nt] = h
            xs[nt].append(h)
        x = new_x
    outs = []
    for nt in _NODE_TYPES:
        outs.append(_jk_project(xs[nt], params['Wout'][nt], params['bout'][nt]))
    return jnp.concatenate(outs, axis=0)


# speedup vs baseline: 9.7289x; 9.7289x over previous
import functools
import numpy as np
import jax
import jax.numpy as jnp
from jax import lax
from jax.experimental import pallas as pl
from jax.experimental.pallas import tpu as pltpu
from jax.experimental.pallas import tpu_sc as plsc

_H = 4
_HID = 128
_DH = _HID // _H
_L = 3
_N = 50000

_NC = 2
_NS = 16
_NW = _NC * _NS
_CHA = 128
_CHUNK_N = 3200
_CHUNK_ALLOC = 3216
_NPAD = 16 * _CHUNK_N

_DNUMS = lax.GatherDimensionNumbers(
    offset_dims=(), collapsed_slice_dims=(0,), start_index_map=(0,))


def _vgather(v, idx):
    return lax.gather(v, idx[:, None], _DNUMS, slice_sizes=(1,),
                      mode=lax.GatherScatterMode.PROMISE_IN_BOUNDS)


def _mesh():
    return plsc.VectorSubcoreMesh(core_axis_name="c", subcore_axis_name="s",
                                  num_cores=_NC, num_subcores=_NS)


@functools.lru_cache(maxsize=None)
def _bucket_maker(epad, e_real, blk):
    tr = epad // _NS
    nblk = tr // blk
    csz = tr + 16

    @functools.partial(
        pl.kernel,
        out_type=(
            jax.ShapeDtypeStruct((16 * epad,), jnp.int32),
            jax.ShapeDtypeStruct((16 * epad,), jnp.int32),
            jax.ShapeDtypeStruct((_NW, 16), jnp.int32),
        ),
        mesh=_mesh(),
        scratch_types=[
            pltpu.VMEM((csz,), jnp.int32),
            pltpu.VMEM((csz,), jnp.int32),
            pltpu.VMEM((blk,), jnp.int32),
            pltpu.VMEM((blk,), jnp.int32),
            pltpu.VMEM((16,), jnp.int32),
        ],
    )
    def bucket(src_hbm, dst_hbm, bsrc_hbm, bdst_hbm, cnts_hbm,
               cs_src, cs_dst, srcg, dstg, cnt_v):
        cid = lax.axis_index("c")
        tid = lax.axis_index("s")
        w = cid * _NS + tid
        tbase = tid * tr
        lane = lax.iota(jnp.int32, 16)

        def zinit(i, _):
            z = jnp.zeros((16,), jnp.int32)
            cs_src[pl.ds(i * 16, 16)] = z
            cs_dst[pl.ds(i * 16, 16)] = z
            return 0
        lax.fori_loop(0, csz // 16, zinit, 0)

        ns = []
        for cc in range(8):
            lo = (cid * 8 + cc) * _CHUNK_N

            def blk_body(bi, n):
                boff = tbase + bi * blk
                pltpu.sync_copy(dst_hbm.at[pl.ds(boff, blk)], dstg)
                pltpu.sync_copy(src_hbm.at[pl.ds(boff, blk)], srcg)

                def grp_body(j, n2):
                    dv = dstg[pl.ds(j * 16, 16)]
                    sv = srcg[pl.ds(j * 16, 16)]
                    ev = boff + j * 16 + lane
                    mb = ((dv >= lo) & (dv < lo + _CHUNK_N)
                          & (ev < e_real))
                    mi = jnp.where(mb, jnp.int32(1), jnp.int32(0))
                    ps = mi
                    for sh in (1, 2, 4, 8):
                        shf = _vgather(ps, jnp.maximum(lane - sh, 0))
                        ps = jnp.where(lane >= sh, ps + shf, ps)
                    cnt = ps[15]
                    pos = jnp.zeros((16,), jnp.int32)
                    for sh in (8, 4, 2, 1):
                        t = _vgather(ps, jnp.minimum(pos + sh - 1, 15))
                        pos = jnp.where(t < lane + 1, pos + sh, pos)
                    pos = pos & 15
                    cs_dst[pl.ds(n2, 16)] = _vgather(dv, pos)
                    cs_src[pl.ds(n2, 16)] = _vgather(sv, pos)
                    return n2 + cnt

                return lax.fori_loop(0, blk // 16, grp_body, n)

            n = lax.fori_loop(0, nblk, blk_body, jnp.int32(0))
            rb = (w * 8 + cc) * tr
            pltpu.sync_copy(cs_src.at[pl.ds(0, tr)],
                            bsrc_hbm.at[pl.ds(rb, tr)])
            pltpu.sync_copy(cs_dst.at[pl.ds(0, tr)],
                            bdst_hbm.at[pl.ds(rb, tr)])
            ns.append(n)

        cvec = jnp.zeros((16,), jnp.int32)
        for i, nv in enumerate(ns):
            cvec = jnp.where(lane == i, nv, cvec)
        cnt_v[...] = cvec
        pltpu.sync_copy(cnt_v, cnts_hbm.at[w])

    return bucket


@functools.lru_cache(maxsize=None)
def _pass_a_maker(epad):
    tr = epad // _NS

    @functools.partial(
        pl.kernel,
        out_type=(
            jax.ShapeDtypeStruct((16 * epad, 16), jnp.float32),
            jax.ShapeDtypeStruct((_NW, 16), jnp.float32),
        ),
        mesh=_mesh(),
        scratch_types=[
            pltpu.VMEM((_CHA,), jnp.int32),
            pltpu.VMEM((_CHA,), jnp.int32),
            pltpu.VMEM((_CHA, 128), jnp.float32),
            pltpu.VMEM((_CHA, 128), jnp.float32),
            pltpu.VMEM((_CHA, 16), jnp.float32),
            pltpu.VMEM((16,), jnp.float32),
            pltpu.VMEM((16,), jnp.float32),
            pltpu.VMEM((16,), jnp.int32),
            pltpu.SemaphoreType.DMA,
            pltpu.SemaphoreType.DMA,
        ],
    )
    def pass_a(q_hbm, k_hbm, bsrc_hbm, bdst_hbm, cnts_hbm, pscale_hbm,
               scores_hbm, tmax_hbm, ibs, ibd, qrows, krows, sc_out,
               psc_v, max_v, cnt_v, sem1, sem2):
        cid = lax.axis_index("c")
        tid = lax.axis_index("s")
        w = cid * _NS + tid
        pltpu.sync_copy(pscale_hbm, psc_v)
        pscale = psc_v[...]
        pltpu.sync_copy(cnts_hbm.at[w], cnt_v)
        cvec = cnt_v[...]
        lane = lax.iota(jnp.int32, 16)

        def vsum16(v):
            for shft in (8, 4, 2, 1):
                v = v + _vgather(v, lane ^ shft)
            return v

        mx = jnp.full((16,), -3e38, jnp.float32)
        for cc in range(8):
            cnt = cvec[cc]
            rb = (w * 8 + cc) * tr
            nch = (cnt + _CHA - 1) // _CHA

            def chunk_body(ci, mx0):
                off = rb + ci * _CHA
                pltpu.sync_copy(bsrc_hbm.at[pl.ds(off, _CHA)], ibs)
                pltpu.sync_copy(bdst_hbm.at[pl.ds(off, _CHA)], ibd)
                c1 = pltpu.async_copy(k_hbm.at[ibs], krows, sem1)
                c2 = pltpu.async_copy(q_hbm.at[ibd], qrows, sem2)
                c1.wait()
                c2.wait()

                def edge_body(e, mx2):
                    svec = jnp.zeros((16,), jnp.float32)
                    for h in range(_H):
                        p0 = (qrows[e, pl.ds(h * 32, 16)]
                              * krows[e, pl.ds(h * 32, 16)])
                        p1 = (qrows[e, pl.ds(h * 32 + 16, 16)]
                              * krows[e, pl.ds(h * 32 + 16, 16)])
                        sh = vsum16(p0 + p1)
                        svec = jnp.where(lane == h, sh, svec)
                    svec = svec * pscale
                    sc_out[e, :] = svec
                    return jnp.maximum(mx2, svec)

                mx0 = lax.fori_loop(0, _CHA, edge_body, mx0)
                pltpu.sync_copy(sc_out, scores_hbm.at[pl.ds(off, _CHA)])
                return mx0

            mx = lax.fori_loop(0, nch, chunk_body, mx)

        max_v[...] = mx
        pltpu.sync_copy(max_v, tmax_hbm.at[w])

    return pass_a


@functools.lru_cache(maxsize=None)
def _pass_b1_maker(epad):
    tr = epad // _NS

    @functools.partial(
        pl.kernel,
        out_type=(
            jax.ShapeDtypeStruct((16 * epad, 128), jnp.float32),
            jax.ShapeDtypeStruct((16 * epad, 16), jnp.float32),
            jax.ShapeDtypeStruct((16 * epad,), jnp.int32),
        ),
        mesh=_mesh(),
        scratch_types=[
            pltpu.VMEM((128,), jnp.int32),
            pltpu.VMEM((128,), jnp.int32),
            pltpu.VMEM((128,), jnp.int32),
            pltpu.VMEM((128, 16), jnp.float32),
            pltpu.VMEM((128, 128), jnp.float32),
            pltpu.VMEM((128, 128), jnp.float32),
            pltpu.VMEM((128, 16), jnp.float32),
            pltpu.VMEM((16,), jnp.float32),
            pltpu.VMEM((16,), jnp.int32),
            pltpu.SemaphoreType.DMA,
        ],
    )
    def pass_b1(sc_hbm, v_hbm, bsrc_hbm, bdst_hbm, cnts_hbm, mv_hbm,
                wnum_hbm, wden_hbm, sidx_hbm,
                gidx, dstb, scat_idx, sgbuf, vgbuf, rowbuf, denbuf,
                mv_v, cnt_v, sem1):
        cid = lax.axis_index("c")
        tid = lax.axis_index("s")
        w = cid * _NS + tid
        lane = lax.iota(jnp.int32, 16)
        pltpu.sync_copy(mv_hbm, mv_v)
        mv = mv_v[...]
        pltpu.sync_copy(cnts_hbm.at[w], cnt_v)
        cvec = cnt_v[...]

        for cc in range(8):
            lo = (cid * 8 + cc) * _CHUNK_N
            rb = (w * 8 + cc) * tr
            cnt = cvec[cc]

            def batch_body(b, _):
                hb = rb + b * 128
                pltpu.sync_copy(bsrc_hbm.at[pl.ds(hb, 128)], gidx)
                pltpu.sync_copy(bdst_hbm.at[pl.ds(hb, 128)], dstb)
                pltpu.sync_copy(sc_hbm.at[pl.ds(hb, 128)], sgbuf)
                pltpu.async_copy(v_hbm.at[gidx], vgbuf, sem1).wait()
                for j in range(8):
                    gl = b * 128 + j * 16 + lane
                    dv = dstb[pl.ds(j * 16, 16)]
                    sel = jnp.where(gl < cnt, dv - lo, _CHUNK_N + lane)
                    scat_idx[pl.ds(j * 16, 16)] = sel

                def edge_body(e, _2):
                    evec = jnp.exp(sgbuf[e, pl.ds(0, 16)] - mv)
                    denbuf[e, pl.ds(0, 16)] = evec
                    for h in range(_H):
                        ehv = _vgather(evec, jnp.full((16,), h, jnp.int32))
                        for j2 in range(2):
                            c0 = h * 32 + j2 * 16
                            rowbuf[e, pl.ds(c0, 16)] = (
                                vgbuf[e, pl.ds(c0, 16)] * ehv)
                    return 0

                lax.fori_loop(0, 128, edge_body, 0)
                pltpu.sync_copy(rowbuf, wnum_hbm.at[pl.ds(hb, 128)])
                pltpu.sync_copy(denbuf, wden_hbm.at[pl.ds(hb, 128)])
                pltpu.sync_copy(scat_idx, sidx_hbm.at[pl.ds(hb, 128)])
                return 0

            nb = (cnt + 127) // 128
            lax.fori_loop(0, nb, batch_body, 0)

    return pass_b1


@functools.lru_cache(maxsize=None)
def _pass_b2_maker(epad):
    tr = epad // _NS

    @functools.partial(
        pl.kernel,
        out_type=(
            jax.ShapeDtypeStruct((_NPAD, 128), jnp.float32),
            jax.ShapeDtypeStruct((_NPAD, 16), jnp.float32),
        ),
        mesh=_mesh(),
        scratch_types=[
            pltpu.VMEM((128,), jnp.int32),
            pltpu.VMEM((128, 128), jnp.float32),
            pltpu.VMEM((128, 16), jnp.float32),
            pltpu.VMEM((128, 128), jnp.float32),
            pltpu.VMEM((128, 16), jnp.float32),
            pltpu.VMEM((16,), jnp.int32),
            pltpu.VMEM_SHARED((_CHUNK_ALLOC, 128), jnp.float32),
            pltpu.VMEM_SHARED((_CHUNK_ALLOC, 16), jnp.float32),
        ],
    )
    def pass_b2(wnum_hbm, wden_hbm, sidx_hbm, cnts_hbm, z128_hbm, z16_hbm,
                num_hbm, den_hbm,
                scat_idx, rowbuf, denbuf, z128, z16, cnt_v,
                sp_num, sp_den):
        cid = lax.axis_index("c")
        tid = lax.axis_index("s")
        w = cid * _NS + tid
        pltpu.sync_copy(cnts_hbm.at[w], cnt_v)
        cvec = cnt_v[...]
        pltpu.sync_copy(z128_hbm, z128)
        pltpu.sync_copy(z16_hbm, z16)

        for cc in range(8):
            lo = (cid * 8 + cc) * _CHUNK_N
            rb = (w * 8 + cc) * tr
            cnt = cvec[cc]

            zrow = tid * (_CHUNK_ALLOC // _NS)
            pltpu.sync_copy(z128, sp_num.at[pl.ds(zrow, 128)])
            pltpu.sync_copy(z16, sp_den.at[pl.ds(zrow, 128)])
            pltpu.sync_copy(z128.at[pl.ds(0, 73)],
                            sp_num.at[pl.ds(zrow + 128, 73)])
            pltpu.sync_copy(z16.at[pl.ds(0, 73)],
                            sp_den.at[pl.ds(zrow + 128, 73)])
            plsc.subcore_barrier()

            def batch_body(b, _):
                hb = rb + b * 128
                pltpu.sync_copy(sidx_hbm.at[pl.ds(hb, 128)], scat_idx)
                pltpu.sync_copy(wnum_hbm.at[pl.ds(hb, 128)], rowbuf)
                pltpu.sync_copy(wden_hbm.at[pl.ds(hb, 128)], denbuf)
                pltpu.sync_copy(rowbuf, sp_num.at[scat_idx], add=True)
                pltpu.sync_copy(denbuf, sp_den.at[scat_idx], add=True)
                return 0

            nb = (cnt + 127) // 128
            lax.fori_loop(0, nb, batch_body, 0)
            plsc.subcore_barrier()

            wrow = tid * (_CHUNK_N // _NS)
            pltpu.sync_copy(sp_num.at[pl.ds(wrow, 200)],
                            num_hbm.at[pl.ds(lo + wrow, 200)])
            pltpu.sync_copy(sp_den.at[pl.ds(wrow, 200)],
                            den_hbm.at[pl.ds(lo + wrow, 200)])
            plsc.subcore_barrier()

    return pass_b2


def _blockdiag(a):
    w = jnp.zeros((_HID, _HID), jnp.float32)
    for h in range(_H):
        w = w.at[h * _DH:(h + 1) * _DH, h * _DH:(h + 1) * _DH].set(a[h])
    return w


_BLK = 1000


def _prep_body(n_out):
    def body(x_ref, w_ref, *outs):
        x = x_ref[...]
        for i in range(n_out):
            outs[i][...] = jnp.dot(x, w_ref[:, i * 128:(i + 1) * 128],
                                   preferred_element_type=jnp.float32)
    return body


def _prep(x, wcat, n_out):
    n = x.shape[0]
    grid = (n // _BLK,)
    return pl.pallas_call(
        _prep_body(n_out),
        grid=grid,
        in_specs=[
            pl.BlockSpec((_BLK, 128), lambda i: (i, 0)),
            pl.BlockSpec((128, n_out * 128), lambda i: (0, 0)),
        ],
        out_specs=[pl.BlockSpec((_BLK, 128), lambda i: (i, 0))] * n_out,
        out_shape=[jax.ShapeDtypeStruct((n, 128), jnp.float32)] * n_out,
    )(x, wcat)


def _node_body_2(num1, den1, num2, den2, x_ref, wa_ref, ex_ref, beta_ref,
                 h_ref, ps_ref, pss_ref):
    de1 = jnp.dot(den1[...], ex_ref[...], preferred_element_type=jnp.float32)
    de2 = jnp.dot(den2[...], ex_ref[...], preferred_element_type=jnp.float32)
    agg = num1[...] / (de1 + 1e-16) + num2[...] / (de2 + 1e-16)
    o = jnp.dot(jax.nn.gelu(agg), wa_ref[...],
                preferred_element_type=jnp.float32)
    b = beta_ref[0, 0]
    h = b * o + (1.0 - b) * x_ref[...]
    h_ref[...] = h
    i = pl.program_id(0)

    @pl.when(i == 0)
    def _():
        ps_ref[...] = jnp.zeros_like(ps_ref)
        pss_ref[...] = jnp.zeros_like(pss_ref)
    ps_ref[...] += jnp.sum(h, axis=0, keepdims=True)
    pss_ref[...] += jnp.sum(h * h, axis=0, keepdims=True)


def _node_body_1(num1, den1, x_ref, wa_ref, ex_ref, beta_ref,
                 h_ref, ps_ref, pss_ref):
    de1 = jnp.dot(den1[...], ex_ref[...], preferred_element_type=jnp.float32)
    agg = num1[...] / (de1 + 1e-16)
    o = jnp.dot(jax.nn.gelu(agg), wa_ref[...],
                preferred_element_type=jnp.float32)
    b = beta_ref[0, 0]
    h = b * o + (1.0 - b) * x_ref[...]
    h_ref[...] = h
    i = pl.program_id(0)

    @pl.when(i == 0)
    def _():
        ps_ref[...] = jnp.zeros_like(ps_ref)
        pss_ref[...] = jnp.zeros_like(pss_ref)
    ps_ref[...] += jnp.sum(h, axis=0, keepdims=True)
    pss_ref[...] += jnp.sum(h * h, axis=0, keepdims=True)


def _node_update(numdens, x, wa, expand, beta):
    n = x.shape[0]
    nb = n // _BLK
    nd_specs = []
    args = []
    for (num, den) in numdens:
        nd_specs += [pl.BlockSpec((_BLK, 128), lambda i: (i, 0)),
                     pl.BlockSpec((_BLK, 16), lambda i: (i, 0))]
        args += [num, den]
    body = _node_body_2 if len(numdens) == 2 else _node_body_1
    return pl.pallas_call(
        body,
        grid=(nb,),
        in_specs=nd_specs + [
            pl.BlockSpec((_BLK, 128), lambda i: (i, 0)),
            pl.BlockSpec((128, 128), lambda i: (0, 0)),
            pl.BlockSpec((16, 128), lambda i: (0, 0)),
            pl.BlockSpec(memory_space=pltpu.SMEM),
        ],
        out_specs=[
            pl.BlockSpec((_BLK, 128), lambda i: (i, 0)),
            pl.BlockSpec((8, 128), lambda i: (0, 0)),
            pl.BlockSpec((8, 128), lambda i: (0, 0)),
        ],
        out_shape=[
            jax.ShapeDtypeStruct((n, 128), jnp.float32),
            jax.ShapeDtypeStruct((8, 128), jnp.float32),
            jax.ShapeDtypeStruct((8, 128), jnp.float32),
        ],
    )(*args, x, wa, expand, beta)


def _norm_body(h_ref, sc_ref, g_ref, b_ref, o_ref):
    mean = sc_ref[0, 0]
    rstd = sc_ref[0, 1]
    o_ref[...] = (h_ref[...] - mean) * rstd * g_ref[...] + b_ref[...]


def _normalize(h, mean, rstd, gamma, beta):
    n = h.shape[0]
    sc = jnp.stack([mean, rstd]).reshape(1, 2)
    return pl.pallas_call(
        _norm_body,
        grid=(n // _BLK,),
        in_specs=[
            pl.BlockSpec((_BLK, 128), lambda i: (i, 0)),
            pl.BlockSpec(memory_space=pltpu.SMEM),
            pl.BlockSpec((1, 128), lambda i: (0, 0)),
            pl.BlockSpec((1, 128), lambda i: (0, 0)),
        ],
        out_specs=pl.BlockSpec((_BLK, 128), lambda i: (i, 0)),
        out_shape=jax.ShapeDtypeStruct((n, 128), jnp.float32),
    )(h, sc, gamma[None, :], beta[None, :])


def _jk_body(h0_ref, h1_ref, h2_ref, w0_ref, w1_ref, w2_ref, b_ref, o_ref):
    acc = jnp.dot(h0_ref[...], w0_ref[...], preferred_element_type=jnp.float32)
    acc += jnp.dot(h1_ref[...], w1_ref[...], preferred_element_type=jnp.float32)
    acc += jnp.dot(h2_ref[...], w2_ref[...], preferred_element_type=jnp.float32)
    o_ref[...] = acc + b_ref[...]


def _jk_project(hs, Wout, bout):
    n = hs[0].shape[0]
    od = Wout.shape[1]
    w0, w1, w2 = Wout[:_HID], Wout[_HID:2 * _HID], Wout[2 * _HID:]
    return pl.pallas_call(
        _jk_body,
        grid=(n // _BLK,),
        in_specs=[
            pl.BlockSpec((_BLK, _HID), lambda i: (i, 0)),
            pl.BlockSpec((_BLK, _HID), lambda i: (i, 0)),
            pl.BlockSpec((_BLK, _HID), lambda i: (i, 0)),
            pl.BlockSpec((_HID, od), lambda i: (0, 0)),
            pl.BlockSpec((_HID, od), lambda i: (0, 0)),
            pl.BlockSpec((_HID, od), lambda i: (0, 0)),
            pl.BlockSpec((1, od), lambda i: (0, 0)),
        ],
        out_specs=pl.BlockSpec((_BLK, od), lambda i: (i, 0)),
        out_shape=jax.ShapeDtypeStruct((n, od), jnp.float32),
    )(hs[0], hs[1], hs[2], w0, w1, w2, bout[None, :])


def kernel(x_paper, x_author, edge_index_cites, edge_index_writes,
           edge_index_rev_writes, batch_paper, batch_author, params):
    f32 = jnp.float32

    def pad_edges(ei, epad):
        e = ei.shape[1]
        src = jnp.zeros((epad,), jnp.int32).at[:e].set(ei[0])
        dst = jnp.zeros((epad,), jnp.int32).at[:e].set(ei[1])
        return src, dst

    src_c, dst_c = pad_edges(edge_index_cites, 204800)
    src_w, dst_w = pad_edges(edge_index_writes, 106496)
    src_r, dst_r = pad_edges(edge_index_rev_writes, 106496)

    bsrc_c, bdst_c, cn_c = _bucket_maker(204800, 200000, 1600)(src_c, dst_c)
    bsrc_w, bdst_w, cn_w = _bucket_maker(106496, 100000, 1664)(src_w, dst_w)
    bsrc_r, bdst_r, cn_r = _bucket_maker(106496, 100000, 1664)(src_r, dst_r)

    expand = jnp.zeros((16, 128), f32)
    for h in range(_H):
        expand = expand.at[h, h * _DH:(h + 1) * _DH].set(1.0)
    z128 = jnp.zeros((128, 128), f32)
    z16 = jnp.zeros((128, 16), f32)

    pa_big = _pass_a_maker(204800)
    pa_small = _pass_a_maker(106496)
    pb1_big = _pass_b1_maker(204800)
    pb1_small = _pass_b1_maker(106496)
    pb2_big = _pass_b2_maker(204800)
    pb2_small = _pass_b2_maker(106496)

    x = {'paper': x_paper, 'author': x_author}
    xs = {'paper': [], 'author': []}

    for l in range(_L):
        lp = params['layers'][l]
        wcat_p = jnp.concatenate([
            lp['Wq']['paper'],
            lp['Wk']['paper'] @ _blockdiag(lp['a_rel']['cites']),
            lp['Wv']['paper'] @ _blockdiag(lp['m_rel']['cites']),
            lp['Wk']['paper'] @ _blockdiag(lp['a_rel']['rev_writes']),
            lp['Wv']['paper'] @ _blockdiag(lp['m_rel']['rev_writes']),
        ], axis=1)
        wcat_a = jnp.concatenate([
            lp['Wq']['author'],
            lp['Wk']['author'] @ _blockdiag(lp['a_rel']['writes']),
            lp['Wv']['author'] @ _blockdiag(lp['m_rel']['writes']),
        ], axis=1)
        q_p, k_c, v_c, k_r, v_r = _prep(x['paper'], wcat_p, 5)
        q_a, k_w, v_w = _prep(x['author'], wcat_a, 3)

        def pscale(rel):
            return jnp.zeros((16,), f32).at[:_H].set(
                lp['p_rel'][rel] / np.sqrt(_DH))

        sc_c, tm_c = pa_big(q_p, k_c, bsrc_c, bdst_c, cn_c, pscale('cites'))
        sc_w, tm_w = pa_small(q_p, k_w, bsrc_w, bdst_w, cn_w,
                              pscale('writes'))
        sc_r, tm_r = pa_small(q_a, k_r, bsrc_r, bdst_r, cn_r,
                              pscale('rev_writes'))

        wn_c, wd_c, si_c = pb1_big(sc_c, v_c, bsrc_c, bdst_c, cn_c,
                                   jnp.max(tm_c, axis=0))
        wn_w, wd_w, si_w = pb1_small(sc_w, v_w, bsrc_w, bdst_w, cn_w,
                                     jnp.max(tm_w, axis=0))
        wn_r, wd_r, si_r = pb1_small(sc_r, v_r, bsrc_r, bdst_r, cn_r,
                                     jnp.max(tm_r, axis=0))
        num_c, den_c = pb2_big(wn_c, wd_c, si_c, cn_c, z128, z16)
        num_w, den_w = pb2_small(wn_w, wd_w, si_w, cn_w, z128, z16)
        num_r, den_r = pb2_small(wn_r, wd_r, si_r, cn_r, z128, z16)

        new_x = {}
        for nt, numdens in (('paper', [(num_c, den_c), (num_w, den_w)]),
                            ('author', [(num_r, den_r)])):
            beta = jax.nn.sigmoid(lp['skip'][nt]).reshape(1, 1)
            h, ps, pss = _node_update(numdens, x[nt], lp['Wa'][nt],
                                      expand, beta)
            cnt = _N * _HID
            mean = jnp.sum(ps[0]) / cnt
            var = jnp.sum(pss[0]) / cnt - mean * mean
            rstd = 1.0 / jnp.sqrt(var + 1e-5)
            hn = _normalize(h, mean, rstd, lp['gamma'][nt], lp['beta'][nt])
            new_x[nt] = hn
            xs[nt].append(hn)
        x = new_x

    outs = [_jk_project(xs[nt], params['Wout'][nt], params['bout'][nt])
            for nt in ('paper', 'author')]
    return jnp.concatenate(outs, axis=0)
